# baseline (speedup 1.0000x reference)
import functools

import jax
import jax.numpy as jnp
from jax.experimental import pallas as pl
from jax.experimental.pallas import tpu as pltpu

_GEO = [1, 2, 3, 4, 5, 6, 8, 10, 22, 23, 29, 31, 32, 33, 43]
_POS = [9, 16, 17, 20, 27, 30, 36, 42, 48, 49, 50]
_SEM = [7, 11, 12, 13, 14, 15, 18, 19, 21, 24, 25, 26, 28, 34, 35, 37, 38,
        39, 40, 41, 44, 45, 46, 47]

_NEG = -1e30


def _row_sum_tree(e):
    s = e[:, 0:8]
    for j in range(1, 8):
        s = s + e[:, 8 * j:8 * j + 8]
    while s.shape[1] > 1:
        h = s.shape[1] // 2
        s = s[:, :h] + s[:, h:]
    return s


def _obj_head_kernel(logit_ref, score_ref, cls_ref):
    x = logit_ref[...]
    col = jax.lax.broadcasted_iota(jnp.int32, x.shape, 1)
    valid = col < 51
    xm = jnp.where(valid, x, _NEG)
    mall = jnp.max(xm, axis=1, keepdims=True)
    e = jnp.where(valid, jnp.exp(x - mall), 0.0)
    denom = _row_sum_tree(e)
    p = e / denom
    p1 = jnp.where((col >= 1) & valid, p, 0.0)
    m = jnp.max(p1, axis=1, keepdims=True)
    amax = jnp.min(jnp.where(p1 == m, col, 64), axis=1, keepdims=True)
    score_ref[...] = m
    cls_ref[...] = amax


def _rel_head_kernel(cat_ref, labels_ref, exp_ref, s_ref, c_ref):
    x = cat_ref[...]
    e = jnp.exp(x)
    col = jax.lax.broadcasted_iota(jnp.int32, x.shape, 1)
    exp_ref[...] = jnp.where(col < 50, e, 0.0)
    lbl = labels_ref[...]
    outs_s = []
    outs_c = []
    for lo, hi in ((0, 15), (15, 26), (26, 50)):
        seg = jnp.where((col >= lo) & (col < hi), e, 0.0)
        m = jnp.max(seg, axis=1, keepdims=True)
        amax = jnp.min(jnp.where(seg == m, col, 64), axis=1, keepdims=True)
        cls = jnp.sum(jnp.where(col == amax, lbl, 0), axis=1, keepdims=True)
        outs_s.append(m)
        outs_c.append(cls)
    s_ref[...] = jnp.concatenate(outs_s + [jnp.zeros_like(outs_s[0])], axis=1)
    c_ref[...] = jnp.concatenate(outs_c + [jnp.zeros_like(outs_c[0])], axis=1)


def _dense_stage(obj_logit, rel_cat):
    n_obj = obj_logit.shape[0]
    n_rel = rel_cat.shape[0]
    obj_pad = jnp.pad(obj_logit, ((0, 0), (0, 64 - obj_logit.shape[1])))
    blk_o = 2000
    obj_score, obj_cls = pl.pallas_call(
        _obj_head_kernel,
        grid=(n_obj // blk_o,),
        in_specs=[pl.BlockSpec((blk_o, 64), lambda i: (i, 0))],
        out_specs=[pl.BlockSpec((blk_o, 1), lambda i: (i, 0)),
                   pl.BlockSpec((blk_o, 1), lambda i: (i, 0))],
        out_shape=[jax.ShapeDtypeStruct((n_obj, 1), jnp.float32),
                   jax.ShapeDtypeStruct((n_obj, 1), jnp.int32)],
    )(obj_pad)

    labels = jnp.array(_GEO + _POS + _SEM + [0] * 14, dtype=jnp.int32)[None, :]
    blk_r = 4000
    rel_exp, rel_s, rel_c = pl.pallas_call(
        _rel_head_kernel,
        grid=(n_rel // blk_r,),
        in_specs=[pl.BlockSpec((blk_r, 64), lambda i: (i, 0)),
                  pl.BlockSpec((1, 64), lambda i: (0, 0))],
        out_specs=[pl.BlockSpec((blk_r, 64), lambda i: (i, 0)),
                   pl.BlockSpec((blk_r, 4), lambda i: (i, 0)),
                   pl.BlockSpec((blk_r, 4), lambda i: (i, 0))],
        out_shape=[jax.ShapeDtypeStruct((n_rel, 64), jnp.float32),
                   jax.ShapeDtypeStruct((n_rel, 4), jnp.float32),
                   jax.ShapeDtypeStruct((n_rel, 4), jnp.int32)],
    )(jnp.pad(rel_cat, ((0, 0), (0, 64 - rel_cat.shape[1])),
              constant_values=_NEG), labels)
    return (obj_score[:, 0], obj_cls[:, 0], rel_exp, rel_s[:, :3], rel_c[:, :3])


def kernel(rel1_prob, rel2_prob, rel3_prob, super_rel_prob, obj_logit,
           rel_pair_idx, boxes):
    del super_rel_prob
    n_rel = rel1_prob.shape[0]
    rel_cat = jnp.concatenate((rel1_prob, rel2_prob, rel3_prob), axis=1)
    obj_score, obj_cls, rel_exp, rel_s, rel_c = _dense_stage(obj_logit, rel_cat)

    obj_class = obj_cls.astype(rel_pair_idx.dtype)

    s0 = obj_score[rel_pair_idx[:, 0]]
    s1 = obj_score[rel_pair_idx[:, 1]]
    triple = ((rel_s * s0[:, None]) * s1[:, None]).T.reshape(-1)
    sorting_idx = jnp.argsort(-triple)
    row = sorting_idx % n_rel
    head = sorting_idx // n_rel
    rel_pair_idx_sorted = rel_pair_idx[row]
    rel_labels_sorted = rel_c[row, head].astype(rel_pair_idx.dtype)
    rel_class_prob_sorted = rel_exp[row, :50]

    return (boxes, obj_class, obj_score, rel_pair_idx_sorted,
            rel_class_prob_sorted, rel_labels_sorted)


# speedup vs baseline: 76.0784x; 76.0784x over previous
import dataclasses

import jax
import jax.numpy as jnp
from jax import lax
from jax.experimental import pallas as pl
from jax.experimental.pallas import tpu as pltpu
from jax.experimental.pallas import tpu_sc as plsc

_GEO = [1, 2, 3, 4, 5, 6, 8, 10, 22, 23, 29, 31, 32, 33, 43]
_POS = [9, 16, 17, 20, 27, 30, 36, 42, 48, 49, 50]
_SEM = [7, 11, 12, 13, 14, 15, 18, 19, 21, 24, 25, 26, 28, 34, 35, 37, 38,
        39, 40, 41, 44, 45, 46, 47]

_NEG = -1e30

N_OBJ = 20000
N_REL = 100000
NP = 100352
NW = 32
CHA = NP // NW
K3 = 3 * NP
NT = 16
CHS = K3 // NT
NBAT = CHS // 128
NB = 2048
NOUT = 3 * N_REL
NOPAD = 300032
NCH = NOPAD // 128

_SC_MESH = plsc.VectorSubcoreMesh(core_axis_name="c", subcore_axis_name="s")
_CP = pltpu.CompilerParams()
if "needs_layout_passes" in pltpu.CompilerParams.__dataclass_fields__:
    _CP = dataclasses.replace(_CP, needs_layout_passes=False)


def _row_sum_tree(e):
    s = e[:, 0:8]
    for j in range(1, 8):
        s = s + e[:, 8 * j:8 * j + 8]
    while s.shape[1] > 1:
        h = s.shape[1] // 2
        s = s[:, :h] + s[:, h:]
    return s


def _obj_head_kernel(logit_ref, score_ref, cls_ref):
    x = logit_ref[...]
    col = lax.broadcasted_iota(jnp.int32, x.shape, 1)
    valid = col < 51
    xm = jnp.where(valid, x, _NEG)
    mall = jnp.max(xm, axis=1, keepdims=True)
    e = jnp.where(valid, jnp.exp(x - mall), 0.0)
    denom = _row_sum_tree(e)
    p = e / denom
    p1 = jnp.where((col >= 1) & valid, p, 0.0)
    m = jnp.max(p1, axis=1, keepdims=True)
    amax = jnp.min(jnp.where(p1 == m, col, 64), axis=1, keepdims=True)
    score_ref[...] = m
    cls_ref[...] = amax


def _rel_head_kernel(cat_ref, labels_ref, exp_ref, s_ref, c_ref):
    x = cat_ref[...]
    e = jnp.exp(x)
    col = lax.broadcasted_iota(jnp.int32, x.shape, 1)
    exp_ref[...] = jnp.where(col < 50, e, 0.0)
    lbl = labels_ref[...]
    outs_s = []
    outs_c = []
    for lo, hi in ((0, 15), (15, 26), (26, 50)):
        seg = jnp.where((col >= lo) & (col < hi), e, 0.0)
        m = jnp.max(seg, axis=1, keepdims=True)
        amax = jnp.min(jnp.where(seg == m, col, 128), axis=1, keepdims=True)
        cls = jnp.sum(jnp.where(col == amax, lbl, 0), axis=1, keepdims=True)
        outs_s.append(m)
        outs_c.append(cls)
    s_ref[...] = jnp.concatenate(outs_s + [jnp.zeros_like(outs_s[0])], axis=1)
    c_ref[...] = jnp.concatenate(outs_c + [jnp.zeros_like(outs_c[0])], axis=1)


def _slice_kernel(src_ref, dst_ref):
    dst_ref[...] = src_ref[...][:, :50]


def _keys_kernel(obj_hbm, pair0_hbm, pair1_hbm, rs_hbm, keys_hbm,
                 objtab, p0v, p1v, rs0v, rs1v, rs2v, k0v, k1v, k2v):
    cid = lax.axis_index("c")
    sid = lax.axis_index("s")
    wid = sid * 2 + cid
    base = wid * CHA
    pltpu.sync_copy(obj_hbm, objtab)
    pltpu.sync_copy(pair0_hbm.at[pl.ds(base, CHA)], p0v)
    pltpu.sync_copy(pair1_hbm.at[pl.ds(base, CHA)], p1v)
    pltpu.sync_copy(rs_hbm.at[pl.ds(0 * NP + base, CHA)], rs0v)
    pltpu.sync_copy(rs_hbm.at[pl.ds(1 * NP + base, CHA)], rs1v)
    pltpu.sync_copy(rs_hbm.at[pl.ds(2 * NP + base, CHA)], rs2v)

    @pl.loop(0, CHA, step=16)
    def _(i):
        s0 = plsc.load_gather(objtab, [p0v[pl.ds(i, 16)]])
        s1 = plsc.load_gather(objtab, [p1v[pl.ds(i, 16)]])
        for rsv, kv in ((rs0v, k0v), (rs1v, k1v), (rs2v, k2v)):
            key = (rsv[pl.ds(i, 16)] * s0) * s1
            bits = plsc.bitcast(key, jnp.int32)
            kv[pl.ds(i, 16)] = ~bits

    pltpu.sync_copy(k0v, keys_hbm.at[pl.ds(0 * NP + base, CHA)])
    pltpu.sync_copy(k1v, keys_hbm.at[pl.ds(1 * NP + base, CHA)])
    pltpu.sync_copy(k2v, keys_hbm.at[pl.ds(2 * NP + base, CHA)])


def _build_keys(obj_score, pair0, pair1, rs_flat):
    return pl.kernel(
        _keys_kernel,
        out_type=jax.ShapeDtypeStruct((K3,), jnp.int32),
        mesh=_SC_MESH,
        compiler_params=_CP,
        scratch_types=[
            pltpu.VMEM((N_OBJ,), jnp.float32),
            pltpu.VMEM((CHA,), jnp.int32),
            pltpu.VMEM((CHA,), jnp.int32),
            pltpu.VMEM((CHA,), jnp.float32),
            pltpu.VMEM((CHA,), jnp.float32),
            pltpu.VMEM((CHA,), jnp.float32),
            pltpu.VMEM((CHA,), jnp.int32),
            pltpu.VMEM((CHA,), jnp.int32),
            pltpu.VMEM((CHA,), jnp.int32),
        ],
    )(obj_score, pair0, pair1, rs_flat)


_SHIFTS = (0, 11, 22)
_MASKS = (2047, 2047, 1023)


def _sort_kernel(keys_hbm, vals_hbm,
                 va, vb, ghist_sp,
                 keys_v, vals_v, posb, hist_v, offs_v, rowbuf_v, totals_v):
    cid = lax.axis_index("c")
    sid = lax.axis_index("s")

    @pl.when(cid == 0)
    def _():
        tid = sid
        base = tid * CHS

        for p in range(3):
            shift = _SHIFTS[p]
            mask = _MASKS[p]
            src_v = (None, va, vb)[p]
            dst_v = (va, vb, va)[p]

            if p == 0:
                pltpu.sync_copy(keys_hbm.at[pl.ds(base, CHS)], keys_v)

                @pl.loop(0, CHS, step=16)
                def _(i):
                    vals_v[pl.ds(i, 16)] = base + i + lax.iota(jnp.int32, 16)
            else:
                pltpu.sync_copy(src_v.at[pl.ds(base, CHS)], vals_v)

                @pl.loop(0, NBAT)
                def _(c):
                    pltpu.sync_copy(
                        keys_hbm.at[vals_v.at[pl.ds(c * 128, 128)]],
                        keys_v.at[pl.ds(c * 128, 128)])

            @pl.loop(0, NB, step=16)
            def _(i):
                hist_v[pl.ds(i, 16)] = jnp.zeros((16,), jnp.int32)

            @pl.loop(0, CHS, step=16)
            def _(i):
                k = plsc.bitcast(keys_v[pl.ds(i, 16)], jnp.uint32)
                d = ((k >> shift) & mask).astype(jnp.int32)
                cnt, last = plsc.scan_count(d)
                plsc.addupdate_scatter(hist_v, [d], cnt, mask=last)

            pltpu.sync_copy(hist_v, ghist_sp.at[tid])
            plsc.subcore_barrier()

            @pl.loop(0, NB, step=16)
            def _(i):
                totals_v[pl.ds(i, 16)] = jnp.zeros((16,), jnp.int32)
                offs_v[pl.ds(i, 16)] = jnp.zeros((16,), jnp.int32)

            for t in range(NT):
                pltpu.sync_copy(ghist_sp.at[t], rowbuf_v)

                @pl.loop(0, NB, step=16)
                def _(i):
                    h = rowbuf_v[pl.ds(i, 16)]
                    totals_v[pl.ds(i, 16)] = totals_v[pl.ds(i, 16)] + h
                    offs_v[pl.ds(i, 16)] = offs_v[pl.ds(i, 16)] + \
                        jnp.where(jnp.full((16,), t, jnp.int32) < tid, h, 0)

            def _scan_body(j, carry):
                v = totals_v[pl.ds(j * 16, 16)]
                cum = plsc.cumsum(v)
                offs_v[pl.ds(j * 16, 16)] = (offs_v[pl.ds(j * 16, 16)]
                                             + cum - v + carry)
                return carry + jnp.sum(v)

            lax.fori_loop(0, NB // 16, _scan_body, jnp.int32(0))

            @pl.loop(0, NBAT)
            def _(c):
                for j in range(8):
                    sl = pl.ds(c * 128 + j * 16, 16)
                    k = plsc.bitcast(keys_v[sl], jnp.uint32)
                    d = ((k >> shift) & mask).astype(jnp.int32)
                    cnt, last = plsc.scan_count(d)
                    bs = plsc.load_gather(offs_v, [d])
                    posb[0, pl.ds(j * 16, 16)] = bs + cnt - 1
                    plsc.addupdate_scatter(offs_v, [d], cnt, mask=last)
                pltpu.sync_copy(vals_v.at[pl.ds(c * 128, 128)],
                                dst_v.at[posb.at[0]])

            plsc.subcore_barrier()

        pltpu.sync_copy(va.at[pl.ds(base, CHS)],
                        vals_hbm.at[pl.ds(base, CHS)])


def _sort(keys):
    return pl.kernel(
        _sort_kernel,
        out_type=jax.ShapeDtypeStruct((K3,), jnp.int32),
        mesh=_SC_MESH,
        compiler_params=_CP,
        scratch_types=[
            pltpu.VMEM_SHARED((K3,), jnp.int32),
            pltpu.VMEM_SHARED((K3,), jnp.int32),
            pltpu.VMEM_SHARED((NT, NB), jnp.int32),
            pltpu.VMEM((CHS,), jnp.int32),
            pltpu.VMEM((CHS,), jnp.int32),
            pltpu.VMEM((1, 128), jnp.int32),
            pltpu.VMEM((NB,), jnp.int32),
            pltpu.VMEM((NB,), jnp.int32),
            pltpu.VMEM((NB,), jnp.int32),
            pltpu.VMEM((NB,), jnp.int32),
        ],
    )(keys)


def _reorder_kernel(vals_hbm, pair0_hbm, pair1_hbm, labels_hbm, exp_hbm,
                    pairs_out, labels_out, probs_out,
                    vbuf, rbuf, lbuf, abuf, bbuf, pbuf, rows_v):
    cid = lax.axis_index("c")
    sid = lax.axis_index("s")
    wid = sid * 2 + cid
    count = (NCH - 1 - wid) // NW + 1

    def _body(i, carry):
        c = wid + i * NW
        pltpu.sync_copy(vals_hbm.at[pl.ds(c * 128, 128)], vbuf.at[0])

        @pl.loop(0, 128, step=16)
        def _(j):
            v = vbuf[0, pl.ds(j, 16)]
            r = jnp.where(v >= NP, v - NP, v)
            r = jnp.where(v >= 2 * NP, r - NP, r)
            rbuf[0, pl.ds(j, 16)] = r

        pltpu.sync_copy(labels_hbm.at[vbuf.at[0]], lbuf)
        pltpu.sync_copy(lbuf, labels_out.at[pl.ds(c * 128, 128)])
        pltpu.sync_copy(pair0_hbm.at[rbuf.at[0]], abuf)
        pltpu.sync_copy(pair1_hbm.at[rbuf.at[0]], bbuf)

        @pl.loop(0, 128, step=16)
        def _(j):
            rr = lax.iota(jnp.int32, 16) + j
            plsc.store_scatter(pbuf, [rr, jnp.zeros((16,), jnp.int32)],
                               abuf[pl.ds(j, 16)])
            plsc.store_scatter(pbuf, [rr, jnp.ones((16,), jnp.int32)],
                               bbuf[pl.ds(j, 16)])

        pltpu.sync_copy(pbuf, pairs_out.at[pl.ds(c * 128, 128)])
        pltpu.sync_copy(exp_hbm.at[rbuf.at[0]], rows_v)
        pltpu.sync_copy(rows_v, probs_out.at[pl.ds(c * 128, 128)])
        return carry

    lax.fori_loop(0, count, _body, jnp.int32(0))


def _reorder(vals, pair0, pair1, labels_flat, exp_tab):
    return pl.kernel(
        _reorder_kernel,
        out_type=(jax.ShapeDtypeStruct((NOPAD, 2), jnp.int32),
                  jax.ShapeDtypeStruct((NOPAD,), jnp.int32),
                  jax.ShapeDtypeStruct((NOPAD, 128), jnp.float32)),
        mesh=_SC_MESH,
        compiler_params=_CP,
        scratch_types=[
            pltpu.VMEM((1, 128), jnp.int32),
            pltpu.VMEM((1, 128), jnp.int32),
            pltpu.VMEM((128,), jnp.int32),
            pltpu.VMEM((128,), jnp.int32),
            pltpu.VMEM((128,), jnp.int32),
            pltpu.VMEM((128, 2), jnp.int32),
            pltpu.VMEM((128, 128), jnp.float32),
        ],
    )(vals, pair0, pair1, labels_flat, exp_tab)


def kernel(rel1_prob, rel2_prob, rel3_prob, super_rel_prob, obj_logit,
           rel_pair_idx, boxes):
    del super_rel_prob
    idt = rel_pair_idx.dtype

    obj_pad = jnp.pad(obj_logit, ((0, 0), (0, 64 - obj_logit.shape[1])))
    blk_o = 2000
    obj_score, obj_cls = pl.pallas_call(
        _obj_head_kernel,
        grid=(N_OBJ // blk_o,),
        in_specs=[pl.BlockSpec((blk_o, 64), lambda i: (i, 0))],
        out_specs=[pl.BlockSpec((blk_o, 1), lambda i: (i, 0)),
                   pl.BlockSpec((blk_o, 1), lambda i: (i, 0))],
        out_shape=[jax.ShapeDtypeStruct((N_OBJ, 1), jnp.float32),
                   jax.ShapeDtypeStruct((N_OBJ, 1), jnp.int32)],
    )(obj_pad)
    obj_score = obj_score[:, 0]

    rel_cat = jnp.concatenate((rel1_prob, rel2_prob, rel3_prob), axis=1)
    rel_cat = jnp.pad(rel_cat, ((0, NP - N_REL), (0, 128 - rel_cat.shape[1])),
                      constant_values=_NEG)
    labels = jnp.array(_GEO + _POS + _SEM + [0] * 78, dtype=jnp.int32)[None, :]
    blk_r = 3136
    rel_exp, rel_s, rel_c = pl.pallas_call(
        _rel_head_kernel,
        grid=(NP // blk_r,),
        in_specs=[pl.BlockSpec((blk_r, 128), lambda i: (i, 0)),
                  pl.BlockSpec((1, 128), lambda i: (0, 0))],
        out_specs=[pl.BlockSpec((blk_r, 128), lambda i: (i, 0)),
                   pl.BlockSpec((blk_r, 4), lambda i: (i, 0)),
                   pl.BlockSpec((blk_r, 4), lambda i: (i, 0))],
        out_shape=[jax.ShapeDtypeStruct((NP, 128), jnp.float32),
                   jax.ShapeDtypeStruct((NP, 4), jnp.float32),
                   jax.ShapeDtypeStruct((NP, 4), jnp.int32)],
    )(rel_cat, labels)

    pair_pad = jnp.pad(rel_pair_idx.astype(jnp.int32),
                       ((0, NP - N_REL), (0, 0)))
    pair0 = pair_pad[:, 0]
    pair1 = pair_pad[:, 1]
    rs_flat = rel_s[:, :3].T.reshape(-1)
    labels_flat = rel_c[:, :3].T.reshape(-1)

    keys = _build_keys(obj_score, pair0, pair1, rs_flat)
    vals = _sort(keys)
    pairs_s, labels_s, probs128 = _reorder(vals, pair0, pair1, labels_flat,
                                           rel_exp)

    probs = pl.pallas_call(
        _slice_kernel,
        grid=(NOPAD // 1024,),
        in_specs=[pl.BlockSpec((1024, 128), lambda i: (i, 0))],
        out_specs=pl.BlockSpec((1024, 50), lambda i: (i, 0)),
        out_shape=jax.ShapeDtypeStruct((NOPAD, 50), jnp.float32),
    )(probs128)

    return (boxes,
            obj_cls[:, 0].astype(idt),
            obj_score,
            pairs_s[:NOUT].astype(idt),
            probs[:NOUT],
            labels_s[:NOUT].astype(idt))


# speedup vs baseline: 86.2174x; 1.1333x over previous
import dataclasses

import jax
import jax.numpy as jnp
from jax import lax
from jax.experimental import pallas as pl
from jax.experimental.pallas import tpu as pltpu
from jax.experimental.pallas import tpu_sc as plsc

_GEO = [1, 2, 3, 4, 5, 6, 8, 10, 22, 23, 29, 31, 32, 33, 43]
_POS = [9, 16, 17, 20, 27, 30, 36, 42, 48, 49, 50]
_SEM = [7, 11, 12, 13, 14, 15, 18, 19, 21, 24, 25, 26, 28, 34, 35, 37, 38,
        39, 40, 41, 44, 45, 46, 47]

_NEG = -1e30

N_OBJ = 20000
N_REL = 100000
NP = 100352
NW = 32
CHA = NP // NW
K3 = 3 * NP
NT = 16
CHS = K3 // NT
NBAT = CHS // 128
NB = 2048
NOUT = 3 * N_REL
NOPAD = 300032
NCH = NOPAD // 128

_SC_MESH = plsc.VectorSubcoreMesh(core_axis_name="c", subcore_axis_name="s")
_CP = pltpu.CompilerParams()
if "needs_layout_passes" in pltpu.CompilerParams.__dataclass_fields__:
    _CP = dataclasses.replace(_CP, needs_layout_passes=False)


def _row_sum_tree(e):
    s = e[:, 0:8]
    for j in range(1, 8):
        s = s + e[:, 8 * j:8 * j + 8]
    while s.shape[1] > 1:
        h = s.shape[1] // 2
        s = s[:, :h] + s[:, h:]
    return s


def _obj_head_kernel(logit_ref, score_ref, cls_ref):
    x = logit_ref[...]
    col = lax.broadcasted_iota(jnp.int32, x.shape, 1)
    valid = col < 51
    xm = jnp.where(valid, x, _NEG)
    mall = jnp.max(xm, axis=1, keepdims=True)
    e = jnp.where(valid, jnp.exp(x - mall), 0.0)
    denom = _row_sum_tree(e)
    p = e / denom
    p1 = jnp.where((col >= 1) & valid, p, 0.0)
    m = jnp.max(p1, axis=1, keepdims=True)
    amax = jnp.min(jnp.where(p1 == m, col, 64), axis=1, keepdims=True)
    score_ref[...] = m
    cls_ref[...] = amax


def _rel_head_kernel(cat_ref, labels_ref, exp_ref, s_ref, c_ref):
    x = cat_ref[...]
    e = jnp.exp(x)
    col = lax.broadcasted_iota(jnp.int32, x.shape, 1)
    exp_ref[...] = jnp.where(col < 50, e, 0.0)
    lbl = labels_ref[...]
    outs_s = []
    outs_c = []
    for lo, hi in ((0, 15), (15, 26), (26, 50)):
        seg = jnp.where((col >= lo) & (col < hi), e, 0.0)
        m = jnp.max(seg, axis=1, keepdims=True)
        amax = jnp.min(jnp.where(seg == m, col, 128), axis=1, keepdims=True)
        cls = jnp.sum(jnp.where(col == amax, lbl, 0), axis=1, keepdims=True)
        outs_s.append(m)
        outs_c.append(cls)
    s_ref[...] = jnp.concatenate(outs_s + [jnp.zeros_like(outs_s[0])], axis=1)
    c_ref[...] = jnp.concatenate(outs_c + [jnp.zeros_like(outs_c[0])], axis=1)


def _slice_kernel(src_ref, dst_ref):
    dst_ref[...] = src_ref[...][:, :50]


def _keys_kernel(obj_hbm, pair0_hbm, pair1_hbm, rs_hbm, keys_hbm,
                 objtab, p0v, p1v, rs0v, rs1v, rs2v, k0v, k1v, k2v):
    cid = lax.axis_index("c")
    sid = lax.axis_index("s")
    wid = sid * 2 + cid
    base = wid * CHA
    pltpu.sync_copy(obj_hbm, objtab)
    pltpu.sync_copy(pair0_hbm.at[pl.ds(base, CHA)], p0v)
    pltpu.sync_copy(pair1_hbm.at[pl.ds(base, CHA)], p1v)
    pltpu.sync_copy(rs_hbm.at[pl.ds(0 * NP + base, CHA)], rs0v)
    pltpu.sync_copy(rs_hbm.at[pl.ds(1 * NP + base, CHA)], rs1v)
    pltpu.sync_copy(rs_hbm.at[pl.ds(2 * NP + base, CHA)], rs2v)

    @pl.loop(0, CHA, step=16)
    def _(i):
        s0 = plsc.load_gather(objtab, [p0v[pl.ds(i, 16)]])
        s1 = plsc.load_gather(objtab, [p1v[pl.ds(i, 16)]])
        for rsv, kv in ((rs0v, k0v), (rs1v, k1v), (rs2v, k2v)):
            key = (rsv[pl.ds(i, 16)] * s0) * s1
            bits = plsc.bitcast(key, jnp.int32)
            kv[pl.ds(i, 16)] = ~bits

    pltpu.sync_copy(k0v, keys_hbm.at[pl.ds(0 * NP + base, CHA)])
    pltpu.sync_copy(k1v, keys_hbm.at[pl.ds(1 * NP + base, CHA)])
    pltpu.sync_copy(k2v, keys_hbm.at[pl.ds(2 * NP + base, CHA)])


def _build_keys(obj_score, pair0, pair1, rs_flat):
    return pl.kernel(
        _keys_kernel,
        out_type=jax.ShapeDtypeStruct((K3,), jnp.int32),
        mesh=_SC_MESH,
        compiler_params=_CP,
        scratch_types=[
            pltpu.VMEM((N_OBJ,), jnp.float32),
            pltpu.VMEM((CHA,), jnp.int32),
            pltpu.VMEM((CHA,), jnp.int32),
            pltpu.VMEM((CHA,), jnp.float32),
            pltpu.VMEM((CHA,), jnp.float32),
            pltpu.VMEM((CHA,), jnp.float32),
            pltpu.VMEM((CHA,), jnp.int32),
            pltpu.VMEM((CHA,), jnp.int32),
            pltpu.VMEM((CHA,), jnp.int32),
        ],
    )(obj_score, pair0, pair1, rs_flat)


_SHIFTS = (0, 11, 22)
_MASKS = (2047, 2047, 1023)


def _sort_kernel(keys_hbm, vals_hbm,
                 va, vb, ghist_sp,
                 keys_v, vals_v, posb, hist_v, offs_v, rowbuf_v, totals_v,
                 dsem):
    cid = lax.axis_index("c")
    sid = lax.axis_index("s")

    @pl.when(cid == 0)
    def _():
        tid = sid
        base = tid * CHS

        for p in range(3):
            shift = _SHIFTS[p]
            mask = _MASKS[p]
            src_v = (None, va, vb)[p]
            dst_v = (va, vb, va)[p]

            if p == 0:
                pltpu.sync_copy(keys_hbm.at[pl.ds(base, CHS)], keys_v)

                @pl.loop(0, CHS, step=16)
                def _(i):
                    vals_v[pl.ds(i, 16)] = base + i + lax.iota(jnp.int32, 16)
            else:
                pltpu.sync_copy(src_v.at[pl.ds(base, CHS)], vals_v)

                @pl.loop(0, NBAT)
                def _(c):
                    pltpu.async_copy(
                        keys_hbm.at[vals_v.at[pl.ds(c * 128, 128)]],
                        keys_v.at[pl.ds(c * 128, 128)], dsem)

                @pl.loop(0, NBAT)
                def _(c):
                    pltpu.make_async_copy(
                        keys_hbm.at[vals_v.at[pl.ds(c * 128, 128)]],
                        keys_v.at[pl.ds(c * 128, 128)], dsem).wait()

            @pl.loop(0, NB, step=16)
            def _(i):
                hist_v[pl.ds(i, 16)] = jnp.zeros((16,), jnp.int32)

            @pl.loop(0, CHS, step=16)
            def _(i):
                k = plsc.bitcast(keys_v[pl.ds(i, 16)], jnp.uint32)
                d = ((k >> shift) & mask).astype(jnp.int32)
                cnt, last = plsc.scan_count(d)
                plsc.addupdate_scatter(hist_v, [d], cnt, mask=last)

            pltpu.sync_copy(hist_v, ghist_sp.at[tid])
            plsc.subcore_barrier()

            @pl.loop(0, NB, step=16)
            def _(i):
                totals_v[pl.ds(i, 16)] = jnp.zeros((16,), jnp.int32)
                offs_v[pl.ds(i, 16)] = jnp.zeros((16,), jnp.int32)

            for t in range(NT):
                pltpu.sync_copy(ghist_sp.at[t], rowbuf_v)

                @pl.loop(0, NB, step=16)
                def _(i):
                    h = rowbuf_v[pl.ds(i, 16)]
                    totals_v[pl.ds(i, 16)] = totals_v[pl.ds(i, 16)] + h
                    offs_v[pl.ds(i, 16)] = offs_v[pl.ds(i, 16)] + \
                        jnp.where(jnp.full((16,), t, jnp.int32) < tid, h, 0)

            def _scan_body(j, carry):
                v = totals_v[pl.ds(j * 16, 16)]
                cum = plsc.cumsum(v)
                offs_v[pl.ds(j * 16, 16)] = (offs_v[pl.ds(j * 16, 16)]
                                             + cum - v + carry)
                return carry + jnp.sum(v)

            lax.fori_loop(0, NB // 16, _scan_body, jnp.int32(0))

            @pl.loop(0, NBAT)
            def _(c):
                for j in range(8):
                    sl = pl.ds(c * 128 + j * 16, 16)
                    k = plsc.bitcast(keys_v[sl], jnp.uint32)
                    d = ((k >> shift) & mask).astype(jnp.int32)
                    cnt, last = plsc.scan_count(d)
                    bs = plsc.load_gather(offs_v, [d])
                    posb[c, pl.ds(j * 16, 16)] = bs + cnt - 1
                    plsc.addupdate_scatter(offs_v, [d], cnt, mask=last)
                pltpu.async_copy(vals_v.at[pl.ds(c * 128, 128)],
                                 dst_v.at[posb.at[c]], dsem)

            @pl.loop(0, NBAT)
            def _(c):
                pltpu.make_async_copy(vals_v.at[pl.ds(c * 128, 128)],
                                      dst_v.at[posb.at[c]], dsem).wait()

            plsc.subcore_barrier()

        pltpu.sync_copy(va.at[pl.ds(base, CHS)],
                        vals_hbm.at[pl.ds(base, CHS)])


def _sort(keys):
    return pl.kernel(
        _sort_kernel,
        out_type=jax.ShapeDtypeStruct((K3,), jnp.int32),
        mesh=_SC_MESH,
        compiler_params=_CP,
        scratch_types=[
            pltpu.VMEM_SHARED((K3,), jnp.int32),
            pltpu.VMEM_SHARED((K3,), jnp.int32),
            pltpu.VMEM_SHARED((NT, NB), jnp.int32),
            pltpu.VMEM((CHS,), jnp.int32),
            pltpu.VMEM((CHS,), jnp.int32),
            pltpu.VMEM((NBAT, 128), jnp.int32),
            pltpu.VMEM((NB,), jnp.int32),
            pltpu.VMEM((NB,), jnp.int32),
            pltpu.VMEM((NB,), jnp.int32),
            pltpu.VMEM((NB,), jnp.int32),
            pltpu.SemaphoreType.DMA,
        ],
    )(keys)


def _reorder_kernel(vals_hbm, pair0_hbm, pair1_hbm, labels_hbm, exp_hbm,
                    pairs_out, labels_out, probs_out,
                    vbuf, rbuf, lbuf, abuf, bbuf, pbuf, rows_v):
    cid = lax.axis_index("c")
    sid = lax.axis_index("s")
    wid = sid * 2 + cid
    count = (NCH - 1 - wid) // NW + 1

    def _body(i, carry):
        c = wid + i * NW
        pltpu.sync_copy(vals_hbm.at[pl.ds(c * 128, 128)], vbuf.at[0])

        @pl.loop(0, 128, step=16)
        def _(j):
            v = vbuf[0, pl.ds(j, 16)]
            r = jnp.where(v >= NP, v - NP, v)
            r = jnp.where(v >= 2 * NP, r - NP, r)
            rbuf[0, pl.ds(j, 16)] = r

        pltpu.sync_copy(labels_hbm.at[vbuf.at[0]], lbuf)
        pltpu.sync_copy(lbuf, labels_out.at[pl.ds(c * 128, 128)])
        pltpu.sync_copy(pair0_hbm.at[rbuf.at[0]], abuf)
        pltpu.sync_copy(pair1_hbm.at[rbuf.at[0]], bbuf)

        @pl.loop(0, 128, step=16)
        def _(j):
            rr = lax.iota(jnp.int32, 16) + j
            plsc.store_scatter(pbuf, [rr, jnp.zeros((16,), jnp.int32)],
                               abuf[pl.ds(j, 16)])
            plsc.store_scatter(pbuf, [rr, jnp.ones((16,), jnp.int32)],
                               bbuf[pl.ds(j, 16)])

        pltpu.sync_copy(pbuf, pairs_out.at[pl.ds(c * 128, 128)])
        pltpu.sync_copy(exp_hbm.at[rbuf.at[0]], rows_v)
        pltpu.sync_copy(rows_v, probs_out.at[pl.ds(c * 128, 128)])
        return carry

    lax.fori_loop(0, count, _body, jnp.int32(0))


def _reorder(vals, pair0, pair1, labels_flat, exp_tab):
    return pl.kernel(
        _reorder_kernel,
        out_type=(jax.ShapeDtypeStruct((NOPAD, 2), jnp.int32),
                  jax.ShapeDtypeStruct((NOPAD,), jnp.int32),
                  jax.ShapeDtypeStruct((NOPAD, 128), jnp.float32)),
        mesh=_SC_MESH,
        compiler_params=_CP,
        scratch_types=[
            pltpu.VMEM((1, 128), jnp.int32),
            pltpu.VMEM((1, 128), jnp.int32),
            pltpu.VMEM((128,), jnp.int32),
            pltpu.VMEM((128,), jnp.int32),
            pltpu.VMEM((128,), jnp.int32),
            pltpu.VMEM((128, 2), jnp.int32),
            pltpu.VMEM((128, 128), jnp.float32),
        ],
    )(vals, pair0, pair1, labels_flat, exp_tab)


def kernel(rel1_prob, rel2_prob, rel3_prob, super_rel_prob, obj_logit,
           rel_pair_idx, boxes):
    del super_rel_prob
    idt = rel_pair_idx.dtype

    obj_pad = jnp.pad(obj_logit, ((0, 0), (0, 64 - obj_logit.shape[1])))
    blk_o = 2000
    obj_score, obj_cls = pl.pallas_call(
        _obj_head_kernel,
        grid=(N_OBJ // blk_o,),
        in_specs=[pl.BlockSpec((blk_o, 64), lambda i: (i, 0))],
        out_specs=[pl.BlockSpec((blk_o, 1), lambda i: (i, 0)),
                   pl.BlockSpec((blk_o, 1), lambda i: (i, 0))],
        out_shape=[jax.ShapeDtypeStruct((N_OBJ, 1), jnp.float32),
                   jax.ShapeDtypeStruct((N_OBJ, 1), jnp.int32)],
    )(obj_pad)
    obj_score = obj_score[:, 0]

    rel_cat = jnp.concatenate((rel1_prob, rel2_prob, rel3_prob), axis=1)
    rel_cat = jnp.pad(rel_cat, ((0, NP - N_REL), (0, 128 - rel_cat.shape[1])),
                      constant_values=_NEG)
    labels = jnp.array(_GEO + _POS + _SEM + [0] * 78, dtype=jnp.int32)[None, :]
    blk_r = 3136
    rel_exp, rel_s, rel_c = pl.pallas_call(
        _rel_head_kernel,
        grid=(NP // blk_r,),
        in_specs=[pl.BlockSpec((blk_r, 128), lambda i: (i, 0)),
                  pl.BlockSpec((1, 128), lambda i: (0, 0))],
        out_specs=[pl.BlockSpec((blk_r, 128), lambda i: (i, 0)),
                   pl.BlockSpec((blk_r, 4), lambda i: (i, 0)),
                   pl.BlockSpec((blk_r, 4), lambda i: (i, 0))],
        out_shape=[jax.ShapeDtypeStruct((NP, 128), jnp.float32),
                   jax.ShapeDtypeStruct((NP, 4), jnp.float32),
                   jax.ShapeDtypeStruct((NP, 4), jnp.int32)],
    )(rel_cat, labels)

    pair_pad = jnp.pad(rel_pair_idx.astype(jnp.int32),
                       ((0, NP - N_REL), (0, 0)))
    pair0 = pair_pad[:, 0]
    pair1 = pair_pad[:, 1]
    rs_flat = rel_s[:, :3].T.reshape(-1)
    labels_flat = rel_c[:, :3].T.reshape(-1)

    keys = _build_keys(obj_score, pair0, pair1, rs_flat)
    vals = _sort(keys)
    pairs_s, labels_s, probs128 = _reorder(vals, pair0, pair1, labels_flat,
                                           rel_exp)

    probs = pl.pallas_call(
        _slice_kernel,
        grid=(NOPAD // 1024,),
        in_specs=[pl.BlockSpec((1024, 128), lambda i: (i, 0))],
        out_specs=pl.BlockSpec((1024, 50), lambda i: (i, 0)),
        out_shape=jax.ShapeDtypeStruct((NOPAD, 50), jnp.float32),
    )(probs128)

    return (boxes,
            obj_cls[:, 0].astype(idt),
            obj_score,
            pairs_s[:NOUT].astype(idt),
            probs[:NOUT],
            labels_s[:NOUT].astype(idt))


# speedup vs baseline: 98.0377x; 1.1371x over previous
import dataclasses

import jax
import jax.numpy as jnp
from jax import lax
from jax.experimental import pallas as pl
from jax.experimental.pallas import tpu as pltpu
from jax.experimental.pallas import tpu_sc as plsc

_GEO = [1, 2, 3, 4, 5, 6, 8, 10, 22, 23, 29, 31, 32, 33, 43]
_POS = [9, 16, 17, 20, 27, 30, 36, 42, 48, 49, 50]
_SEM = [7, 11, 12, 13, 14, 15, 18, 19, 21, 24, 25, 26, 28, 34, 35, 37, 38,
        39, 40, 41, 44, 45, 46, 47]

_NEG = -1e30

N_OBJ = 20000
N_REL = 100000
NP = 100352
NW = 32
CHA = NP // NW
K3 = 3 * NP
NT = 16
CHS = K3 // NT
NBAT = CHS // 128
NB = 2048
NOUT = 3 * N_REL
NOPAD = 300032
NCH = NOPAD // 128

_SC_MESH = plsc.VectorSubcoreMesh(core_axis_name="c", subcore_axis_name="s")
_CP = pltpu.CompilerParams()
if "needs_layout_passes" in pltpu.CompilerParams.__dataclass_fields__:
    _CP = dataclasses.replace(_CP, needs_layout_passes=False)


def _row_sum_tree(e):
    s = e[:, 0:8]
    for j in range(1, 8):
        s = s + e[:, 8 * j:8 * j + 8]
    while s.shape[1] > 1:
        h = s.shape[1] // 2
        s = s[:, :h] + s[:, h:]
    return s


def _obj_head_kernel(logit_ref, score_ref, cls_ref):
    x = logit_ref[...]
    col = lax.broadcasted_iota(jnp.int32, x.shape, 1)
    valid = col < 51
    xm = jnp.where(valid, x, _NEG)
    mall = jnp.max(xm, axis=1, keepdims=True)
    e = jnp.where(valid, jnp.exp(x - mall), 0.0)
    denom = _row_sum_tree(e)
    p = e / denom
    p1 = jnp.where((col >= 1) & valid, p, 0.0)
    m = jnp.max(p1, axis=1, keepdims=True)
    amax = jnp.min(jnp.where(p1 == m, col, 64), axis=1, keepdims=True)
    score_ref[...] = m
    cls_ref[...] = amax


def _rel_head_kernel(cat_ref, labels_ref, exp_ref, s_ref, c_ref):
    x = cat_ref[...]
    e = jnp.exp(x)
    col = lax.broadcasted_iota(jnp.int32, x.shape, 1)
    exp_ref[...] = jnp.where(col < 50, e, 0.0)
    lbl = labels_ref[...]
    outs_s = []
    outs_c = []
    for lo, hi in ((0, 15), (15, 26), (26, 50)):
        seg = jnp.where((col >= lo) & (col < hi), e, 0.0)
        m = jnp.max(seg, axis=1, keepdims=True)
        amax = jnp.min(jnp.where(seg == m, col, 128), axis=1, keepdims=True)
        cls = jnp.sum(jnp.where(col == amax, lbl, 0), axis=1, keepdims=True)
        outs_s.append(m)
        outs_c.append(cls)
    s_ref[...] = jnp.concatenate(outs_s + [jnp.zeros_like(outs_s[0])], axis=1)
    c_ref[...] = jnp.concatenate(outs_c + [jnp.zeros_like(outs_c[0])], axis=1)


def _slice_kernel(src_ref, dst_ref):
    dst_ref[...] = src_ref[...][:, :50]


def _keys_kernel(obj_hbm, pair0_hbm, pair1_hbm, rs_hbm, keys_hbm,
                 objtab, p0v, p1v, rs0v, rs1v, rs2v, k0v, k1v, k2v):
    cid = lax.axis_index("c")
    sid = lax.axis_index("s")
    wid = sid * 2 + cid
    base = wid * CHA
    pltpu.sync_copy(obj_hbm, objtab)
    pltpu.sync_copy(pair0_hbm.at[pl.ds(base, CHA)], p0v)
    pltpu.sync_copy(pair1_hbm.at[pl.ds(base, CHA)], p1v)
    pltpu.sync_copy(rs_hbm.at[pl.ds(0 * NP + base, CHA)], rs0v)
    pltpu.sync_copy(rs_hbm.at[pl.ds(1 * NP + base, CHA)], rs1v)
    pltpu.sync_copy(rs_hbm.at[pl.ds(2 * NP + base, CHA)], rs2v)

    @pl.loop(0, CHA, step=16)
    def _(i):
        s0 = plsc.load_gather(objtab, [p0v[pl.ds(i, 16)]])
        s1 = plsc.load_gather(objtab, [p1v[pl.ds(i, 16)]])
        for rsv, kv in ((rs0v, k0v), (rs1v, k1v), (rs2v, k2v)):
            key = (rsv[pl.ds(i, 16)] * s0) * s1
            bits = plsc.bitcast(key, jnp.int32)
            kv[pl.ds(i, 16)] = ~bits

    pltpu.sync_copy(k0v, keys_hbm.at[pl.ds(0 * NP + base, CHA)])
    pltpu.sync_copy(k1v, keys_hbm.at[pl.ds(1 * NP + base, CHA)])
    pltpu.sync_copy(k2v, keys_hbm.at[pl.ds(2 * NP + base, CHA)])


def _build_keys(obj_score, pair0, pair1, rs_flat):
    return pl.kernel(
        _keys_kernel,
        out_type=jax.ShapeDtypeStruct((K3,), jnp.int32),
        mesh=_SC_MESH,
        compiler_params=_CP,
        scratch_types=[
            pltpu.VMEM((N_OBJ,), jnp.float32),
            pltpu.VMEM((CHA,), jnp.int32),
            pltpu.VMEM((CHA,), jnp.int32),
            pltpu.VMEM((CHA,), jnp.float32),
            pltpu.VMEM((CHA,), jnp.float32),
            pltpu.VMEM((CHA,), jnp.float32),
            pltpu.VMEM((CHA,), jnp.int32),
            pltpu.VMEM((CHA,), jnp.int32),
            pltpu.VMEM((CHA,), jnp.int32),
        ],
    )(obj_score, pair0, pair1, rs_flat)


_SHIFTS = (0, 11, 22)
_MASKS = (2047, 2047, 1023)


def _sort_kernel(keys_hbm, vals_hbm,
                 va, vb, ghist_sp,
                 keys_v, vals_v, posb, hist_v, offs_v, rowbuf_v, totals_v,
                 dsem):
    cid = lax.axis_index("c")
    sid = lax.axis_index("s")

    @pl.when(cid == 0)
    def _():
        tid = sid
        base = tid * CHS

        for p in range(3):
            shift = _SHIFTS[p]
            mask = _MASKS[p]
            src_v = (None, va, vb)[p]
            dst_v = (va, vb, va)[p]

            if p == 0:
                pltpu.sync_copy(keys_hbm.at[pl.ds(base, CHS)], keys_v)

                @pl.loop(0, CHS, step=16)
                def _(i):
                    vals_v[pl.ds(i, 16)] = base + i + lax.iota(jnp.int32, 16)
            else:
                pltpu.sync_copy(src_v.at[pl.ds(base, CHS)], vals_v)

                @pl.loop(0, NBAT)
                def _(c):
                    pltpu.async_copy(
                        keys_hbm.at[vals_v.at[pl.ds(c * 128, 128)]],
                        keys_v.at[pl.ds(c * 128, 128)], dsem)

                @pl.loop(0, NBAT)
                def _(c):
                    pltpu.make_async_copy(
                        keys_hbm.at[vals_v.at[pl.ds(c * 128, 128)]],
                        keys_v.at[pl.ds(c * 128, 128)], dsem).wait()

            @pl.loop(0, NB, step=16)
            def _(i):
                hist_v[pl.ds(i, 16)] = jnp.zeros((16,), jnp.int32)

            @pl.loop(0, CHS, step=16)
            def _(i):
                k = plsc.bitcast(keys_v[pl.ds(i, 16)], jnp.uint32)
                d = ((k >> shift) & mask).astype(jnp.int32)
                cnt, last = plsc.scan_count(d)
                plsc.addupdate_scatter(hist_v, [d], cnt, mask=last)

            pltpu.sync_copy(hist_v, ghist_sp.at[tid])
            plsc.subcore_barrier()

            @pl.loop(0, NB, step=16)
            def _(i):
                totals_v[pl.ds(i, 16)] = jnp.zeros((16,), jnp.int32)
                offs_v[pl.ds(i, 16)] = jnp.zeros((16,), jnp.int32)

            for t in range(NT):
                pltpu.sync_copy(ghist_sp.at[t], rowbuf_v)

                @pl.loop(0, NB, step=16)
                def _(i):
                    h = rowbuf_v[pl.ds(i, 16)]
                    totals_v[pl.ds(i, 16)] = totals_v[pl.ds(i, 16)] + h
                    offs_v[pl.ds(i, 16)] = offs_v[pl.ds(i, 16)] + \
                        jnp.where(jnp.full((16,), t, jnp.int32) < tid, h, 0)

            def _scan_body(j, carry):
                v = totals_v[pl.ds(j * 16, 16)]
                cum = plsc.cumsum(v)
                offs_v[pl.ds(j * 16, 16)] = (offs_v[pl.ds(j * 16, 16)]
                                             + cum - v + carry)
                return carry + jnp.sum(v)

            lax.fori_loop(0, NB // 16, _scan_body, jnp.int32(0))

            @pl.loop(0, NBAT)
            def _(c):
                for j in range(8):
                    sl = pl.ds(c * 128 + j * 16, 16)
                    k = plsc.bitcast(keys_v[sl], jnp.uint32)
                    d = ((k >> shift) & mask).astype(jnp.int32)
                    cnt, last = plsc.scan_count(d)
                    bs = plsc.load_gather(offs_v, [d])
                    posb[c, pl.ds(j * 16, 16)] = bs + cnt - 1
                    plsc.addupdate_scatter(offs_v, [d], cnt, mask=last)
                pltpu.async_copy(vals_v.at[pl.ds(c * 128, 128)],
                                 dst_v.at[posb.at[c]], dsem)

            @pl.loop(0, NBAT)
            def _(c):
                pltpu.make_async_copy(vals_v.at[pl.ds(c * 128, 128)],
                                      dst_v.at[posb.at[c]], dsem).wait()

            plsc.subcore_barrier()

        pltpu.sync_copy(va.at[pl.ds(base, CHS)],
                        vals_hbm.at[pl.ds(base, CHS)])


def _sort(keys):
    return pl.kernel(
        _sort_kernel,
        out_type=jax.ShapeDtypeStruct((K3,), jnp.int32),
        mesh=_SC_MESH,
        compiler_params=_CP,
        scratch_types=[
            pltpu.VMEM_SHARED((K3,), jnp.int32),
            pltpu.VMEM_SHARED((K3,), jnp.int32),
            pltpu.VMEM_SHARED((NT, NB), jnp.int32),
            pltpu.VMEM((CHS,), jnp.int32),
            pltpu.VMEM((CHS,), jnp.int32),
            pltpu.VMEM((NBAT, 128), jnp.int32),
            pltpu.VMEM((NB,), jnp.int32),
            pltpu.VMEM((NB,), jnp.int32),
            pltpu.VMEM((NB,), jnp.int32),
            pltpu.VMEM((NB,), jnp.int32),
            pltpu.SemaphoreType.DMA,
        ],
    )(keys)


def _reorder_kernel(vals_hbm, pair0_hbm, pair1_hbm, labels_hbm, exp_hbm,
                    pairs_out, labels_out, probs_out,
                    vbuf, rbuf, lbuf, abuf, bbuf, pbuf, rows_v, gsem, osem):
    cid = lax.axis_index("c")
    sid = lax.axis_index("s")
    wid = sid * 2 + cid
    count = (NCH - 1 - wid) // NW + 1

    def _out_descs(c):
        return (pltpu.make_async_copy(lbuf, labels_out.at[pl.ds(c * 128, 128)],
                                      osem),
                pltpu.make_async_copy(pbuf, pairs_out.at[pl.ds(c * 128, 128)],
                                      osem),
                pltpu.make_async_copy(rows_v, probs_out.at[pl.ds(c * 128, 128)],
                                      osem))

    def _body(i, carry):
        c = wid + i * NW
        pltpu.sync_copy(vals_hbm.at[pl.ds(c * 128, 128)], vbuf.at[0])

        @pl.loop(0, 128, step=16)
        def _(j):
            v = vbuf[0, pl.ds(j, 16)]
            r = jnp.where(v >= NP, v - NP, v)
            r = jnp.where(v >= 2 * NP, r - NP, r)
            rbuf[0, pl.ds(j, 16)] = r

        @pl.when(i > 0)
        def _():
            for dsc in _out_descs(c - NW):
                dsc.wait()

        pltpu.async_copy(labels_hbm.at[vbuf.at[0]], lbuf, gsem)
        pltpu.async_copy(pair0_hbm.at[rbuf.at[0]], abuf, gsem)
        pltpu.async_copy(pair1_hbm.at[rbuf.at[0]], bbuf, gsem)
        pltpu.async_copy(exp_hbm.at[rbuf.at[0]], rows_v, gsem)
        pltpu.make_async_copy(labels_hbm.at[vbuf.at[0]], lbuf, gsem).wait()
        pltpu.make_async_copy(pair0_hbm.at[rbuf.at[0]], abuf, gsem).wait()
        pltpu.make_async_copy(pair1_hbm.at[rbuf.at[0]], bbuf, gsem).wait()
        pltpu.make_async_copy(exp_hbm.at[rbuf.at[0]], rows_v, gsem).wait()

        @pl.loop(0, 128, step=16)
        def _(j):
            rr = lax.iota(jnp.int32, 16) + j
            plsc.store_scatter(pbuf, [rr, jnp.zeros((16,), jnp.int32)],
                               abuf[pl.ds(j, 16)])
            plsc.store_scatter(pbuf, [rr, jnp.ones((16,), jnp.int32)],
                               bbuf[pl.ds(j, 16)])

        for dsc in _out_descs(c):
            dsc.start()
        return carry

    lax.fori_loop(0, count, _body, jnp.int32(0))
    for dsc in _out_descs(wid + (count - 1) * NW):
        dsc.wait()


def _reorder(vals, pair0, pair1, labels_flat, exp_tab):
    return pl.kernel(
        _reorder_kernel,
        out_type=(jax.ShapeDtypeStruct((NOPAD, 2), jnp.int32),
                  jax.ShapeDtypeStruct((NOPAD,), jnp.int32),
                  jax.ShapeDtypeStruct((NOPAD, 128), jnp.float32)),
        mesh=_SC_MESH,
        compiler_params=_CP,
        scratch_types=[
            pltpu.VMEM((1, 128), jnp.int32),
            pltpu.VMEM((1, 128), jnp.int32),
            pltpu.VMEM((128,), jnp.int32),
            pltpu.VMEM((128,), jnp.int32),
            pltpu.VMEM((128,), jnp.int32),
            pltpu.VMEM((128, 2), jnp.int32),
            pltpu.VMEM((128, 128), jnp.float32),
            pltpu.SemaphoreType.DMA,
            pltpu.SemaphoreType.DMA,
        ],
    )(vals, pair0, pair1, labels_flat, exp_tab)


def kernel(rel1_prob, rel2_prob, rel3_prob, super_rel_prob, obj_logit,
           rel_pair_idx, boxes):
    del super_rel_prob
    idt = rel_pair_idx.dtype

    obj_pad = jnp.pad(obj_logit, ((0, 0), (0, 64 - obj_logit.shape[1])))
    blk_o = 2000
    obj_score, obj_cls = pl.pallas_call(
        _obj_head_kernel,
        grid=(N_OBJ // blk_o,),
        in_specs=[pl.BlockSpec((blk_o, 64), lambda i: (i, 0))],
        out_specs=[pl.BlockSpec((blk_o, 1), lambda i: (i, 0)),
                   pl.BlockSpec((blk_o, 1), lambda i: (i, 0))],
        out_shape=[jax.ShapeDtypeStruct((N_OBJ, 1), jnp.float32),
                   jax.ShapeDtypeStruct((N_OBJ, 1), jnp.int32)],
    )(obj_pad)
    obj_score = obj_score[:, 0]

    rel_cat = jnp.concatenate((rel1_prob, rel2_prob, rel3_prob), axis=1)
    rel_cat = jnp.pad(rel_cat, ((0, NP - N_REL), (0, 128 - rel_cat.shape[1])),
                      constant_values=_NEG)
    labels = jnp.array(_GEO + _POS + _SEM + [0] * 78, dtype=jnp.int32)[None, :]
    blk_r = 3136
    rel_exp, rel_s, rel_c = pl.pallas_call(
        _rel_head_kernel,
        grid=(NP // blk_r,),
        in_specs=[pl.BlockSpec((blk_r, 128), lambda i: (i, 0)),
                  pl.BlockSpec((1, 128), lambda i: (0, 0))],
        out_specs=[pl.BlockSpec((blk_r, 128), lambda i: (i, 0)),
                   pl.BlockSpec((blk_r, 4), lambda i: (i, 0)),
                   pl.BlockSpec((blk_r, 4), lambda i: (i, 0))],
        out_shape=[jax.ShapeDtypeStruct((NP, 128), jnp.float32),
                   jax.ShapeDtypeStruct((NP, 4), jnp.float32),
                   jax.ShapeDtypeStruct((NP, 4), jnp.int32)],
    )(rel_cat, labels)

    pair_pad = jnp.pad(rel_pair_idx.astype(jnp.int32),
                       ((0, NP - N_REL), (0, 0)))
    pair0 = pair_pad[:, 0]
    pair1 = pair_pad[:, 1]
    rs_flat = rel_s[:, :3].T.reshape(-1)
    labels_flat = rel_c[:, :3].T.reshape(-1)

    keys = _build_keys(obj_score, pair0, pair1, rs_flat)
    vals = _sort(keys)
    pairs_s, labels_s, probs128 = _reorder(vals, pair0, pair1, labels_flat,
                                           rel_exp)

    probs = pl.pallas_call(
        _slice_kernel,
        grid=(NOPAD // 1024,),
        in_specs=[pl.BlockSpec((1024, 128), lambda i: (i, 0))],
        out_specs=pl.BlockSpec((1024, 50), lambda i: (i, 0)),
        out_shape=jax.ShapeDtypeStruct((NOPAD, 50), jnp.float32),
    )(probs128)

    return (boxes,
            obj_cls[:, 0].astype(idt),
            obj_score,
            pairs_s[:NOUT].astype(idt),
            probs[:NOUT],
            labels_s[:NOUT].astype(idt))


# speedup vs baseline: 98.0544x; 1.0002x over previous
import dataclasses

import jax
import jax.numpy as jnp
from jax import lax
from jax.experimental import pallas as pl
from jax.experimental.pallas import tpu as pltpu
from jax.experimental.pallas import tpu_sc as plsc

_GEO = [1, 2, 3, 4, 5, 6, 8, 10, 22, 23, 29, 31, 32, 33, 43]
_POS = [9, 16, 17, 20, 27, 30, 36, 42, 48, 49, 50]
_SEM = [7, 11, 12, 13, 14, 15, 18, 19, 21, 24, 25, 26, 28, 34, 35, 37, 38,
        39, 40, 41, 44, 45, 46, 47]

_NEG = -1e30

N_OBJ = 20000
N_REL = 100000
NP = 100352
NW = 32
CHA = NP // NW
K3 = 3 * NP
NT = 16
CHS = K3 // NT
NBAT = CHS // 128
NB = 2048
NOUT = 3 * N_REL
NOPAD = 300032
NCH = NOPAD // 128

_SC_MESH = plsc.VectorSubcoreMesh(core_axis_name="c", subcore_axis_name="s")
_CP = pltpu.CompilerParams()
if "needs_layout_passes" in pltpu.CompilerParams.__dataclass_fields__:
    _CP = dataclasses.replace(_CP, needs_layout_passes=False)


def _row_sum_tree(e):
    s = e[:, 0:8]
    for j in range(1, 8):
        s = s + e[:, 8 * j:8 * j + 8]
    while s.shape[1] > 1:
        h = s.shape[1] // 2
        s = s[:, :h] + s[:, h:]
    return s


def _obj_head_kernel(logit_ref, score_ref, cls_ref):
    x = logit_ref[...]
    col = lax.broadcasted_iota(jnp.int32, x.shape, 1)
    valid = col < 51
    xm = jnp.where(valid, x, _NEG)
    mall = jnp.max(xm, axis=1, keepdims=True)
    e = jnp.where(valid, jnp.exp(x - mall), 0.0)
    denom = _row_sum_tree(e)
    p = e / denom
    p1 = jnp.where((col >= 1) & valid, p, 0.0)
    m = jnp.max(p1, axis=1, keepdims=True)
    amax = jnp.min(jnp.where(p1 == m, col, 64), axis=1, keepdims=True)
    score_ref[...] = m
    cls_ref[...] = amax


def _rel_head_kernel(cat_ref, labels_ref, exp_ref, s_ref, c_ref):
    x = cat_ref[...]
    e = jnp.exp(x)
    col = lax.broadcasted_iota(jnp.int32, x.shape, 1)
    exp_ref[:, :50] = e
    lbl = labels_ref[...]
    outs_s = []
    outs_c = []
    for lo, hi in ((0, 15), (15, 26), (26, 50)):
        seg = jnp.where((col >= lo) & (col < hi), e, 0.0)
        m = jnp.max(seg, axis=1, keepdims=True)
        amax = jnp.min(jnp.where(seg == m, col, 128), axis=1, keepdims=True)
        cls = jnp.sum(jnp.where(col == amax, lbl, 0), axis=1, keepdims=True)
        outs_s.append(m)
        outs_c.append(cls)
    s_ref[...] = jnp.concatenate(outs_s + [jnp.zeros_like(outs_s[0])], axis=1)
    c_ref[...] = jnp.concatenate(outs_c + [jnp.zeros_like(outs_c[0])], axis=1)


def _slice_kernel(src_ref, dst_ref):
    dst_ref[...] = src_ref[...][:, :50]


def _keys_kernel(obj_hbm, pair0_hbm, pair1_hbm, rs_hbm, keys_hbm,
                 objtab, p0v, p1v, rs0v, rs1v, rs2v, k0v, k1v, k2v):
    cid = lax.axis_index("c")
    sid = lax.axis_index("s")
    wid = sid * 2 + cid
    base = wid * CHA
    pltpu.sync_copy(obj_hbm, objtab)
    pltpu.sync_copy(pair0_hbm.at[pl.ds(base, CHA)], p0v)
    pltpu.sync_copy(pair1_hbm.at[pl.ds(base, CHA)], p1v)
    pltpu.sync_copy(rs_hbm.at[pl.ds(0 * NP + base, CHA)], rs0v)
    pltpu.sync_copy(rs_hbm.at[pl.ds(1 * NP + base, CHA)], rs1v)
    pltpu.sync_copy(rs_hbm.at[pl.ds(2 * NP + base, CHA)], rs2v)

    @pl.loop(0, CHA, step=16)
    def _(i):
        s0 = plsc.load_gather(objtab, [p0v[pl.ds(i, 16)]])
        s1 = plsc.load_gather(objtab, [p1v[pl.ds(i, 16)]])
        for rsv, kv in ((rs0v, k0v), (rs1v, k1v), (rs2v, k2v)):
            key = (rsv[pl.ds(i, 16)] * s0) * s1
            bits = plsc.bitcast(key, jnp.int32)
            kv[pl.ds(i, 16)] = ~bits

    pltpu.sync_copy(k0v, keys_hbm.at[pl.ds(0 * NP + base, CHA)])
    pltpu.sync_copy(k1v, keys_hbm.at[pl.ds(1 * NP + base, CHA)])
    pltpu.sync_copy(k2v, keys_hbm.at[pl.ds(2 * NP + base, CHA)])


def _build_keys(obj_score, pair0, pair1, rs_flat):
    return pl.kernel(
        _keys_kernel,
        out_type=jax.ShapeDtypeStruct((K3,), jnp.int32),
        mesh=_SC_MESH,
        compiler_params=_CP,
        scratch_types=[
            pltpu.VMEM((N_OBJ,), jnp.float32),
            pltpu.VMEM((CHA,), jnp.int32),
            pltpu.VMEM((CHA,), jnp.int32),
            pltpu.VMEM((CHA,), jnp.float32),
            pltpu.VMEM((CHA,), jnp.float32),
            pltpu.VMEM((CHA,), jnp.float32),
            pltpu.VMEM((CHA,), jnp.int32),
            pltpu.VMEM((CHA,), jnp.int32),
            pltpu.VMEM((CHA,), jnp.int32),
        ],
    )(obj_score, pair0, pair1, rs_flat)


_SHIFTS = (0, 11, 22)
_MASKS = (2047, 2047, 1023)


def _sort_kernel(keys_hbm, vals_hbm,
                 va, vb, ghist_sp,
                 keys_v, vals_v, posb, hist_v, offs_v, rowbuf_v, totals_v,
                 dsem):
    cid = lax.axis_index("c")
    sid = lax.axis_index("s")

    @pl.when(cid == 0)
    def _():
        tid = sid
        base = tid * CHS

        for p in range(3):
            shift = _SHIFTS[p]
            mask = _MASKS[p]
            src_v = (None, va, vb)[p]
            dst_v = (va, vb, va)[p]

            if p == 0:
                pltpu.sync_copy(keys_hbm.at[pl.ds(base, CHS)], keys_v)

                @pl.loop(0, CHS, step=16)
                def _(i):
                    vals_v[pl.ds(i, 16)] = base + i + lax.iota(jnp.int32, 16)
            else:
                pltpu.sync_copy(src_v.at[pl.ds(base, CHS)], vals_v)

                @pl.loop(0, NBAT)
                def _(c):
                    pltpu.async_copy(
                        keys_hbm.at[vals_v.at[pl.ds(c * 128, 128)]],
                        keys_v.at[pl.ds(c * 128, 128)], dsem)

                @pl.loop(0, NBAT)
                def _(c):
                    pltpu.make_async_copy(
                        keys_hbm.at[vals_v.at[pl.ds(c * 128, 128)]],
                        keys_v.at[pl.ds(c * 128, 128)], dsem).wait()

            @pl.loop(0, NB, step=16)
            def _(i):
                hist_v[pl.ds(i, 16)] = jnp.zeros((16,), jnp.int32)

            @pl.loop(0, CHS, step=16)
            def _(i):
                k = plsc.bitcast(keys_v[pl.ds(i, 16)], jnp.uint32)
                d = ((k >> shift) & mask).astype(jnp.int32)
                cnt, last = plsc.scan_count(d)
                plsc.addupdate_scatter(hist_v, [d], cnt, mask=last)

            pltpu.sync_copy(hist_v, ghist_sp.at[tid])
            plsc.subcore_barrier()

            @pl.loop(0, NB, step=16)
            def _(i):
                totals_v[pl.ds(i, 16)] = jnp.zeros((16,), jnp.int32)
                offs_v[pl.ds(i, 16)] = jnp.zeros((16,), jnp.int32)

            for t in range(NT):
                pltpu.sync_copy(ghist_sp.at[t], rowbuf_v)

                @pl.loop(0, NB, step=16)
                def _(i):
                    h = rowbuf_v[pl.ds(i, 16)]
                    totals_v[pl.ds(i, 16)] = totals_v[pl.ds(i, 16)] + h
                    offs_v[pl.ds(i, 16)] = offs_v[pl.ds(i, 16)] + \
                        jnp.where(jnp.full((16,), t, jnp.int32) < tid, h, 0)

            def _scan_body(j, carry):
                v = totals_v[pl.ds(j * 16, 16)]
                cum = plsc.cumsum(v)
                offs_v[pl.ds(j * 16, 16)] = (offs_v[pl.ds(j * 16, 16)]
                                             + cum - v + carry)
                return carry + jnp.sum(v)

            lax.fori_loop(0, NB // 16, _scan_body, jnp.int32(0))

            @pl.loop(0, NBAT)
            def _(c):
                for j in range(8):
                    sl = pl.ds(c * 128 + j * 16, 16)
                    k = plsc.bitcast(keys_v[sl], jnp.uint32)
                    d = ((k >> shift) & mask).astype(jnp.int32)
                    cnt, last = plsc.scan_count(d)
                    bs = plsc.load_gather(offs_v, [d])
                    posb[c, pl.ds(j * 16, 16)] = bs + cnt - 1
                    plsc.addupdate_scatter(offs_v, [d], cnt, mask=last)
                pltpu.async_copy(vals_v.at[pl.ds(c * 128, 128)],
                                 dst_v.at[posb.at[c]], dsem)

            @pl.loop(0, NBAT)
            def _(c):
                pltpu.make_async_copy(vals_v.at[pl.ds(c * 128, 128)],
                                      dst_v.at[posb.at[c]], dsem).wait()

            plsc.subcore_barrier()

        pltpu.sync_copy(va.at[pl.ds(base, CHS)],
                        vals_hbm.at[pl.ds(base, CHS)])


def _sort(keys):
    return pl.kernel(
        _sort_kernel,
        out_type=jax.ShapeDtypeStruct((K3,), jnp.int32),
        mesh=_SC_MESH,
        compiler_params=_CP,
        scratch_types=[
            pltpu.VMEM_SHARED((K3,), jnp.int32),
            pltpu.VMEM_SHARED((K3,), jnp.int32),
            pltpu.VMEM_SHARED((NT, NB), jnp.int32),
            pltpu.VMEM((CHS,), jnp.int32),
            pltpu.VMEM((CHS,), jnp.int32),
            pltpu.VMEM((NBAT, 128), jnp.int32),
            pltpu.VMEM((NB,), jnp.int32),
            pltpu.VMEM((NB,), jnp.int32),
            pltpu.VMEM((NB,), jnp.int32),
            pltpu.VMEM((NB,), jnp.int32),
            pltpu.SemaphoreType.DMA,
        ],
    )(keys)


def _reorder_kernel(vals_hbm, pair0_hbm, pair1_hbm, labels_hbm, exp_hbm,
                    pairs_out, labels_out, probs_out,
                    vbuf, rbuf, lbuf, abuf, bbuf, pbuf, rows_v, gsem, osem):
    cid = lax.axis_index("c")
    sid = lax.axis_index("s")
    wid = sid * 2 + cid
    count = (NCH - 1 - wid) // NW + 1

    def _out_descs(c):
        return (pltpu.make_async_copy(lbuf, labels_out.at[pl.ds(c * 128, 128)],
                                      osem),
                pltpu.make_async_copy(pbuf, pairs_out.at[pl.ds(c * 128, 128)],
                                      osem),
                pltpu.make_async_copy(rows_v, probs_out.at[pl.ds(c * 128, 128)],
                                      osem))

    def _body(i, carry):
        c = wid + i * NW
        pltpu.sync_copy(vals_hbm.at[pl.ds(c * 128, 128)], vbuf.at[0])

        @pl.loop(0, 128, step=16)
        def _(j):
            v = vbuf[0, pl.ds(j, 16)]
            r = jnp.where(v >= NP, v - NP, v)
            r = jnp.where(v >= 2 * NP, r - NP, r)
            rbuf[0, pl.ds(j, 16)] = jnp.minimum(r, N_REL - 1)

        @pl.when(i > 0)
        def _():
            for dsc in _out_descs(c - NW):
                dsc.wait()

        pltpu.async_copy(labels_hbm.at[vbuf.at[0]], lbuf, gsem)
        pltpu.async_copy(pair0_hbm.at[rbuf.at[0]], abuf, gsem)
        pltpu.async_copy(pair1_hbm.at[rbuf.at[0]], bbuf, gsem)
        pltpu.async_copy(exp_hbm.at[rbuf.at[0]], rows_v, gsem)
        pltpu.make_async_copy(labels_hbm.at[vbuf.at[0]], lbuf, gsem).wait()
        pltpu.make_async_copy(pair0_hbm.at[rbuf.at[0]], abuf, gsem).wait()
        pltpu.make_async_copy(pair1_hbm.at[rbuf.at[0]], bbuf, gsem).wait()
        pltpu.make_async_copy(exp_hbm.at[rbuf.at[0]], rows_v, gsem).wait()

        @pl.loop(0, 128, step=16)
        def _(j):
            rr = lax.iota(jnp.int32, 16) + j
            plsc.store_scatter(pbuf, [rr, jnp.zeros((16,), jnp.int32)],
                               abuf[pl.ds(j, 16)])
            plsc.store_scatter(pbuf, [rr, jnp.ones((16,), jnp.int32)],
                               bbuf[pl.ds(j, 16)])

        for dsc in _out_descs(c):
            dsc.start()
        return carry

    lax.fori_loop(0, count, _body, jnp.int32(0))
    for dsc in _out_descs(wid + (count - 1) * NW):
        dsc.wait()


def _reorder(vals, pair0, pair1, labels_flat, exp_tab):
    return pl.kernel(
        _reorder_kernel,
        out_type=(jax.ShapeDtypeStruct((NOPAD, 2), jnp.int32),
                  jax.ShapeDtypeStruct((NOPAD,), jnp.int32),
                  jax.ShapeDtypeStruct((NOPAD, 128), jnp.float32)),
        mesh=_SC_MESH,
        compiler_params=_CP,
        scratch_types=[
            pltpu.VMEM((1, 128), jnp.int32),
            pltpu.VMEM((1, 128), jnp.int32),
            pltpu.VMEM((128,), jnp.int32),
            pltpu.VMEM((128,), jnp.int32),
            pltpu.VMEM((128,), jnp.int32),
            pltpu.VMEM((128, 2), jnp.int32),
            pltpu.VMEM((128, 128), jnp.float32),
            pltpu.SemaphoreType.DMA,
            pltpu.SemaphoreType.DMA,
        ],
    )(vals, pair0, pair1, labels_flat, exp_tab)


def kernel(rel1_prob, rel2_prob, rel3_prob, super_rel_prob, obj_logit,
           rel_pair_idx, boxes):
    del super_rel_prob
    idt = rel_pair_idx.dtype

    obj_pad = jnp.pad(obj_logit, ((0, 0), (0, 64 - obj_logit.shape[1])))
    blk_o = 2000
    obj_score, obj_cls = pl.pallas_call(
        _obj_head_kernel,
        grid=(N_OBJ // blk_o,),
        in_specs=[pl.BlockSpec((blk_o, 64), lambda i: (i, 0))],
        out_specs=[pl.BlockSpec((blk_o, 1), lambda i: (i, 0)),
                   pl.BlockSpec((blk_o, 1), lambda i: (i, 0))],
        out_shape=[jax.ShapeDtypeStruct((N_OBJ, 1), jnp.float32),
                   jax.ShapeDtypeStruct((N_OBJ, 1), jnp.int32)],
    )(obj_pad)
    obj_score = obj_score[:, 0]

    rel_cat = jnp.concatenate((rel1_prob, rel2_prob, rel3_prob), axis=1)
    labels = jnp.array(_GEO + _POS + _SEM, dtype=jnp.int32)[None, :]
    blk_r = 2000
    rel_exp, rel_s, rel_c = pl.pallas_call(
        _rel_head_kernel,
        grid=(N_REL // blk_r,),
        in_specs=[pl.BlockSpec((blk_r, 50), lambda i: (i, 0)),
                  pl.BlockSpec((1, 50), lambda i: (0, 0))],
        out_specs=[pl.BlockSpec((blk_r, 128), lambda i: (i, 0)),
                   pl.BlockSpec((blk_r, 4), lambda i: (i, 0)),
                   pl.BlockSpec((blk_r, 4), lambda i: (i, 0))],
        out_shape=[jax.ShapeDtypeStruct((N_REL, 128), jnp.float32),
                   jax.ShapeDtypeStruct((N_REL, 4), jnp.float32),
                   jax.ShapeDtypeStruct((N_REL, 4), jnp.int32)],
    )(rel_cat, labels)

    pair_pad = jnp.pad(rel_pair_idx.astype(jnp.int32),
                       ((0, NP - N_REL), (0, 0)))
    pair0 = pair_pad[:, 0]
    pair1 = pair_pad[:, 1]
    rs_flat = jnp.pad(rel_s[:, :3],
                      ((0, NP - N_REL), (0, 0))).T.reshape(-1)
    labels_flat = jnp.pad(rel_c[:, :3],
                          ((0, NP - N_REL), (0, 0))).T.reshape(-1)

    keys = _build_keys(obj_score, pair0, pair1, rs_flat)
    vals = _sort(keys)
    pairs_s, labels_s, probs128 = _reorder(vals, pair0, pair1, labels_flat,
                                           rel_exp)

    probs = pl.pallas_call(
        _slice_kernel,
        grid=(NOPAD // 1024,),
        in_specs=[pl.BlockSpec((1024, 128), lambda i: (i, 0))],
        out_specs=pl.BlockSpec((1024, 50), lambda i: (i, 0)),
        out_shape=jax.ShapeDtypeStruct((NOPAD, 50), jnp.float32),
    )(probs128)

    return (boxes,
            obj_cls[:, 0].astype(idt),
            obj_score,
            pairs_s[:NOUT].astype(idt),
            probs[:NOUT],
            labels_s[:NOUT].astype(idt))


# speedup vs baseline: 115.0575x; 1.1734x over previous
import dataclasses

import jax
import jax.numpy as jnp
from jax import lax
from jax.experimental import pallas as pl
from jax.experimental.pallas import tpu as pltpu
from jax.experimental.pallas import tpu_sc as plsc

_GEO = [1, 2, 3, 4, 5, 6, 8, 10, 22, 23, 29, 31, 32, 33, 43]
_POS = [9, 16, 17, 20, 27, 30, 36, 42, 48, 49, 50]
_SEM = [7, 11, 12, 13, 14, 15, 18, 19, 21, 24, 25, 26, 28, 34, 35, 37, 38,
        39, 40, 41, 44, 45, 46, 47]

_NEG = -1e30

N_OBJ = 20000
N_REL = 100000
NP = 100352
NW = 32
CHA = NP // NW
K3 = 3 * NP
NT = 16
CHS = K3 // NT
NBAT = CHS // 128
NB = 2048
NOUT = 3 * N_REL
NOPAD = 300032
NCH = NOPAD // 128

_SC_MESH = plsc.VectorSubcoreMesh(core_axis_name="c", subcore_axis_name="s")
_CP = pltpu.CompilerParams()
if "needs_layout_passes" in pltpu.CompilerParams.__dataclass_fields__:
    _CP = dataclasses.replace(_CP, needs_layout_passes=False)


def _row_sum_tree(e):
    s = e[:, 0:8]
    for j in range(1, 8):
        s = s + e[:, 8 * j:8 * j + 8]
    while s.shape[1] > 1:
        h = s.shape[1] // 2
        s = s[:, :h] + s[:, h:]
    return s


def _obj_head_kernel(logit_ref, score_ref, cls_ref):
    x = logit_ref[...]
    col = lax.broadcasted_iota(jnp.int32, x.shape, 1)
    valid = col < 51
    xm = jnp.where(valid, x, _NEG)
    mall = jnp.max(xm, axis=1, keepdims=True)
    e = jnp.where(valid, jnp.exp(x - mall), 0.0)
    denom = _row_sum_tree(e)
    p = e / denom
    p1 = jnp.where((col >= 1) & valid, p, 0.0)
    m = jnp.max(p1, axis=1, keepdims=True)
    amax = jnp.min(jnp.where(p1 == m, col, 64), axis=1, keepdims=True)
    score_ref[...] = m
    cls_ref[...] = amax


def _rel_head_kernel(cat_ref, labels_ref, exp_ref, s_ref, c_ref):
    x = cat_ref[...]
    e = jnp.exp(x)
    col = lax.broadcasted_iota(jnp.int32, x.shape, 1)
    exp_ref[:, :50] = e
    lbl = labels_ref[...]
    outs_s = []
    outs_c = []
    for lo, hi in ((0, 15), (15, 26), (26, 50)):
        seg = jnp.where((col >= lo) & (col < hi), e, 0.0)
        m = jnp.max(seg, axis=1, keepdims=True)
        amax = jnp.min(jnp.where(seg == m, col, 128), axis=1, keepdims=True)
        cls = jnp.sum(jnp.where(col == amax, lbl, 0), axis=1, keepdims=True)
        outs_s.append(m)
        outs_c.append(cls)
    s_ref[...] = jnp.concatenate(outs_s + [jnp.zeros_like(outs_s[0])], axis=1)
    c_ref[...] = jnp.concatenate(outs_c + [jnp.zeros_like(outs_c[0])], axis=1)


def _slice_kernel(src_ref, dst_ref):
    dst_ref[...] = src_ref[...][:, :50]


def _keys_kernel(obj_hbm, pair0_hbm, pair1_hbm, rs_hbm, keys_hbm,
                 objtab, p0v, p1v, rs0v, rs1v, rs2v, k0v, k1v, k2v):
    cid = lax.axis_index("c")
    sid = lax.axis_index("s")
    wid = sid * 2 + cid
    base = wid * CHA
    pltpu.sync_copy(obj_hbm, objtab)
    pltpu.sync_copy(pair0_hbm.at[pl.ds(base, CHA)], p0v)
    pltpu.sync_copy(pair1_hbm.at[pl.ds(base, CHA)], p1v)
    pltpu.sync_copy(rs_hbm.at[pl.ds(0 * NP + base, CHA)], rs0v)
    pltpu.sync_copy(rs_hbm.at[pl.ds(1 * NP + base, CHA)], rs1v)
    pltpu.sync_copy(rs_hbm.at[pl.ds(2 * NP + base, CHA)], rs2v)

    @pl.loop(0, CHA, step=16)
    def _(i):
        s0 = plsc.load_gather(objtab, [p0v[pl.ds(i, 16)]])
        s1 = plsc.load_gather(objtab, [p1v[pl.ds(i, 16)]])
        for rsv, kv in ((rs0v, k0v), (rs1v, k1v), (rs2v, k2v)):
            key = (rsv[pl.ds(i, 16)] * s0) * s1
            bits = plsc.bitcast(key, jnp.int32)
            kv[pl.ds(i, 16)] = ~bits

    pltpu.sync_copy(k0v, keys_hbm.at[pl.ds(0 * NP + base, CHA)])
    pltpu.sync_copy(k1v, keys_hbm.at[pl.ds(1 * NP + base, CHA)])
    pltpu.sync_copy(k2v, keys_hbm.at[pl.ds(2 * NP + base, CHA)])


def _build_keys(obj_score, pair0, pair1, rs_flat):
    return pl.kernel(
        _keys_kernel,
        out_type=jax.ShapeDtypeStruct((K3,), jnp.int32),
        mesh=_SC_MESH,
        compiler_params=_CP,
        scratch_types=[
            pltpu.VMEM((N_OBJ,), jnp.float32),
            pltpu.VMEM((CHA,), jnp.int32),
            pltpu.VMEM((CHA,), jnp.int32),
            pltpu.VMEM((CHA,), jnp.float32),
            pltpu.VMEM((CHA,), jnp.float32),
            pltpu.VMEM((CHA,), jnp.float32),
            pltpu.VMEM((CHA,), jnp.int32),
            pltpu.VMEM((CHA,), jnp.int32),
            pltpu.VMEM((CHA,), jnp.int32),
        ],
    )(obj_score, pair0, pair1, rs_flat)


_SHIFTS = (0, 11, 22)
_MASKS = (2047, 2047, 1023)


def _sort_kernel(keys_hbm, vals_hbm,
                 va, vb, ghist_sp,
                 keys_v, vals_v, posb, hist_v, offs_v, rowbuf_v, totals_v,
                 dsem):
    cid = lax.axis_index("c")
    sid = lax.axis_index("s")

    @pl.when(cid == 0)
    def _():
        tid = sid
        base = tid * CHS

        for p in range(3):
            shift = _SHIFTS[p]
            mask = _MASKS[p]
            src_v = (None, va, vb)[p]
            dst_v = (va, vb, va)[p]

            if p == 0:
                pltpu.sync_copy(keys_hbm.at[pl.ds(base, CHS)], keys_v)

                @pl.loop(0, CHS, step=16)
                def _(i):
                    vals_v[pl.ds(i, 16)] = base + i + lax.iota(jnp.int32, 16)
            else:
                pltpu.sync_copy(src_v.at[pl.ds(base, CHS)], vals_v)

                @pl.loop(0, NBAT)
                def _(c):
                    pltpu.async_copy(
                        keys_hbm.at[vals_v.at[pl.ds(c * 128, 128)]],
                        keys_v.at[pl.ds(c * 128, 128)], dsem)

                @pl.loop(0, NBAT)
                def _(c):
                    pltpu.make_async_copy(
                        keys_hbm.at[vals_v.at[pl.ds(c * 128, 128)]],
                        keys_v.at[pl.ds(c * 128, 128)], dsem).wait()

            @pl.loop(0, NB, step=16)
            def _(i):
                hist_v[pl.ds(i, 16)] = jnp.zeros((16,), jnp.int32)

            @pl.loop(0, CHS, step=16)
            def _(i):
                k = plsc.bitcast(keys_v[pl.ds(i, 16)], jnp.uint32)
                d = ((k >> shift) & mask).astype(jnp.int32)
                cnt, last = plsc.scan_count(d)
                plsc.addupdate_scatter(hist_v, [d], cnt, mask=last)

            pltpu.sync_copy(hist_v, ghist_sp.at[tid])
            plsc.subcore_barrier()

            @pl.loop(0, NB, step=16)
            def _(i):
                totals_v[pl.ds(i, 16)] = jnp.zeros((16,), jnp.int32)
                offs_v[pl.ds(i, 16)] = jnp.zeros((16,), jnp.int32)

            for t in range(NT):
                pltpu.sync_copy(ghist_sp.at[t], rowbuf_v)

                @pl.loop(0, NB, step=16)
                def _(i):
                    h = rowbuf_v[pl.ds(i, 16)]
                    totals_v[pl.ds(i, 16)] = totals_v[pl.ds(i, 16)] + h
                    offs_v[pl.ds(i, 16)] = offs_v[pl.ds(i, 16)] + \
                        jnp.where(jnp.full((16,), t, jnp.int32) < tid, h, 0)

            def _scan_body(j, carry):
                v = totals_v[pl.ds(j * 16, 16)]
                cum = plsc.cumsum(v)
                offs_v[pl.ds(j * 16, 16)] = (offs_v[pl.ds(j * 16, 16)]
                                             + cum - v + carry)
                return carry + jnp.sum(v)

            lax.fori_loop(0, NB // 16, _scan_body, jnp.int32(0))

            @pl.loop(0, NBAT)
            def _(c):
                for j in range(8):
                    sl = pl.ds(c * 128 + j * 16, 16)
                    k = plsc.bitcast(keys_v[sl], jnp.uint32)
                    d = ((k >> shift) & mask).astype(jnp.int32)
                    cnt, last = plsc.scan_count(d)
                    bs = plsc.load_gather(offs_v, [d])
                    posb[c, pl.ds(j * 16, 16)] = bs + cnt - 1
                    plsc.addupdate_scatter(offs_v, [d], cnt, mask=last)
                pltpu.async_copy(vals_v.at[pl.ds(c * 128, 128)],
                                 dst_v.at[posb.at[c]], dsem)

            @pl.loop(0, NBAT)
            def _(c):
                pltpu.make_async_copy(vals_v.at[pl.ds(c * 128, 128)],
                                      dst_v.at[posb.at[c]], dsem).wait()

            plsc.subcore_barrier()

        pltpu.sync_copy(va.at[pl.ds(base, CHS)],
                        vals_hbm.at[pl.ds(base, CHS)])


def _sort(keys):
    return pl.kernel(
        _sort_kernel,
        out_type=jax.ShapeDtypeStruct((K3,), jnp.int32),
        mesh=_SC_MESH,
        compiler_params=_CP,
        scratch_types=[
            pltpu.VMEM_SHARED((K3,), jnp.int32),
            pltpu.VMEM_SHARED((K3,), jnp.int32),
            pltpu.VMEM_SHARED((NT, NB), jnp.int32),
            pltpu.VMEM((CHS,), jnp.int32),
            pltpu.VMEM((CHS,), jnp.int32),
            pltpu.VMEM((NBAT, 128), jnp.int32),
            pltpu.VMEM((NB,), jnp.int32),
            pltpu.VMEM((NB,), jnp.int32),
            pltpu.VMEM((NB,), jnp.int32),
            pltpu.VMEM((NB,), jnp.int32),
            pltpu.SemaphoreType.DMA,
        ],
    )(keys)


def _reorder_kernel(vals_hbm, pair0_hbm, pair1_hbm, labels_hbm, exp_hbm,
                    pairs_out, labels_out, probs_out,
                    vbuf, rbuf, lbuf, abuf, bbuf, pbuf, rows_v, gsem, osem):
    cid = lax.axis_index("c")
    sid = lax.axis_index("s")
    wid = sid * 2 + cid
    count = (NCH - 1 - wid) // NW + 1

    def _out_descs(c):
        return (pltpu.make_async_copy(lbuf, labels_out.at[pl.ds(c * 128, 128)],
                                      osem),
                pltpu.make_async_copy(pbuf, pairs_out.at[pl.ds(c * 128, 128)],
                                      osem),
                pltpu.make_async_copy(rows_v, probs_out.at[pl.ds(c * 128, 128)],
                                      osem))

    def _body(i, carry):
        c = wid + i * NW
        pltpu.sync_copy(vals_hbm.at[pl.ds(c * 128, 128)], vbuf.at[0])

        @pl.loop(0, 128, step=16)
        def _(j):
            v = vbuf[0, pl.ds(j, 16)]
            r = jnp.where(v >= NP, v - NP, v)
            r = jnp.where(v >= 2 * NP, r - NP, r)
            rbuf[0, pl.ds(j, 16)] = jnp.minimum(r, N_REL - 1)

        @pl.when(i > 0)
        def _():
            for dsc in _out_descs(c - NW):
                dsc.wait()

        pltpu.async_copy(labels_hbm.at[vbuf.at[0]], lbuf, gsem)
        pltpu.async_copy(pair0_hbm.at[rbuf.at[0]], abuf, gsem)
        pltpu.async_copy(pair1_hbm.at[rbuf.at[0]], bbuf, gsem)
        pltpu.async_copy(exp_hbm.at[rbuf.at[0]], rows_v, gsem)
        pltpu.make_async_copy(labels_hbm.at[vbuf.at[0]], lbuf, gsem).wait()
        pltpu.make_async_copy(pair0_hbm.at[rbuf.at[0]], abuf, gsem).wait()
        pltpu.make_async_copy(pair1_hbm.at[rbuf.at[0]], bbuf, gsem).wait()
        pltpu.make_async_copy(exp_hbm.at[rbuf.at[0]], rows_v, gsem).wait()

        @pl.loop(0, 128, step=16)
        def _(j):
            rr = lax.iota(jnp.int32, 16) + j
            plsc.store_scatter(pbuf, [rr, jnp.zeros((16,), jnp.int32)],
                               abuf[pl.ds(j, 16)])
            plsc.store_scatter(pbuf, [rr, jnp.ones((16,), jnp.int32)],
                               bbuf[pl.ds(j, 16)])

        for dsc in _out_descs(c):
            dsc.start()
        return carry

    lax.fori_loop(0, count, _body, jnp.int32(0))
    for dsc in _out_descs(wid + (count - 1) * NW):
        dsc.wait()


def _reorder(vals, pair0, pair1, labels_flat, exp_tab):
    return pl.kernel(
        _reorder_kernel,
        out_type=(jax.ShapeDtypeStruct((NOPAD, 2), jnp.int32),
                  jax.ShapeDtypeStruct((NOPAD,), jnp.int32),
                  jax.ShapeDtypeStruct((NOPAD, 128), jnp.float32)),
        mesh=_SC_MESH,
        compiler_params=_CP,
        scratch_types=[
            pltpu.VMEM((1, 128), jnp.int32),
            pltpu.VMEM((1, 128), jnp.int32),
            pltpu.VMEM((128,), jnp.int32),
            pltpu.VMEM((128,), jnp.int32),
            pltpu.VMEM((128,), jnp.int32),
            pltpu.VMEM((128, 2), jnp.int32),
            pltpu.VMEM((128, 128), jnp.float32),
            pltpu.SemaphoreType.DMA,
            pltpu.SemaphoreType.DMA,
        ],
    )(vals, pair0, pair1, labels_flat, exp_tab)


def kernel(rel1_prob, rel2_prob, rel3_prob, super_rel_prob, obj_logit,
           rel_pair_idx, boxes):
    del super_rel_prob
    idt = rel_pair_idx.dtype

    obj_pad = jnp.pad(obj_logit, ((0, 0), (0, 64 - obj_logit.shape[1])))
    blk_o = 2000
    obj_score, obj_cls = pl.pallas_call(
        _obj_head_kernel,
        grid=(N_OBJ // blk_o,),
        in_specs=[pl.BlockSpec((blk_o, 64), lambda i: (i, 0))],
        out_specs=[pl.BlockSpec((blk_o, 1), lambda i: (i, 0)),
                   pl.BlockSpec((blk_o, 1), lambda i: (i, 0))],
        out_shape=[jax.ShapeDtypeStruct((N_OBJ, 1), jnp.float32),
                   jax.ShapeDtypeStruct((N_OBJ, 1), jnp.int32)],
    )(obj_pad)
    obj_score = obj_score[:, 0]

    rel_cat = jnp.concatenate((rel1_prob, rel2_prob, rel3_prob), axis=1)
    labels = jnp.array(_GEO + _POS + _SEM, dtype=jnp.int32)[None, :]
    blk_r = 2000
    rel_exp, rel_s, rel_c = pl.pallas_call(
        _rel_head_kernel,
        grid=(N_REL // blk_r,),
        in_specs=[pl.BlockSpec((blk_r, 50), lambda i: (i, 0)),
                  pl.BlockSpec((1, 50), lambda i: (0, 0))],
        out_specs=[pl.BlockSpec((blk_r, 128), lambda i: (i, 0)),
                   pl.BlockSpec((blk_r, 4), lambda i: (i, 0)),
                   pl.BlockSpec((blk_r, 4), lambda i: (i, 0))],
        out_shape=[jax.ShapeDtypeStruct((N_REL, 128), jnp.float32),
                   jax.ShapeDtypeStruct((N_REL, 4), jnp.float32),
                   jax.ShapeDtypeStruct((N_REL, 4), jnp.int32)],
    )(rel_cat, labels)

    pair_pad = jnp.pad(rel_pair_idx.astype(jnp.int32),
                       ((0, NP - N_REL), (0, 0)))
    pair0 = pair_pad[:, 0]
    pair1 = pair_pad[:, 1]
    rs_flat = jnp.pad(rel_s[:, :3],
                      ((0, NP - N_REL), (0, 0))).T.reshape(-1)
    labels_flat = jnp.pad(rel_c[:, :3],
                          ((0, NP - N_REL), (0, 0))).T.reshape(-1)

    keys = _build_keys(obj_score, pair0, pair1, rs_flat)
    vals = _sort(keys)
    pairs_s, labels_s, probs128 = _reorder(vals, pair0, pair1, labels_flat,
                                           rel_exp)

    return (boxes,
            obj_cls[:, 0].astype(idt),
            obj_score,
            pairs_s[:NOUT].astype(idt),
            probs128[:NOUT, :50],
            labels_s[:NOUT].astype(idt))
